# baseline (speedup 1.0000x reference)
import jax
import jax.numpy as jnp
from jax.experimental import pallas as pl
from jax.experimental.pallas import tpu as pltpu

_B, _S, _DIM = 4, 4096, 2048
_E, _TOPK = 16, 2
_AUX_COEF = 0.01
_T = _B * _S
_TBLK = 512
_GRID = _T // _TBLK


def _router_kernel(x_ref, w_ref, idx_ref, gate_ref, aux_ref, cnt_ref, psum_ref):
    i = pl.program_id(0)
    logits = jax.lax.dot_general(
        x_ref[...], w_ref[...],
        dimension_numbers=(((1,), (0,)), ((), ())),
        preferred_element_type=jnp.float32)
    m = jnp.max(logits, axis=-1, keepdims=True)
    ex = jnp.exp(logits - m)
    z = jnp.sum(ex, axis=-1, keepdims=True)
    p = ex / z

    iota = jax.lax.broadcasted_iota(jnp.int32, p.shape, 1)
    m1 = jnp.max(p, axis=-1, keepdims=True)
    idx1 = jnp.min(jnp.where(p >= m1, iota, _E), axis=-1, keepdims=True)
    pm = jnp.where(iota == idx1, -1.0, p)
    m2 = jnp.max(pm, axis=-1, keepdims=True)
    idx2 = jnp.min(jnp.where(pm >= m2, iota, _E), axis=-1, keepdims=True)

    denom = m1 + m2 + 1e-9
    idx_ref[...] = jnp.concatenate([idx1, idx2], axis=-1)
    gate_ref[...] = jnp.concatenate([m1 / denom, m2 / denom], axis=-1)

    cnt = (jnp.sum(jnp.where(iota == idx1, 1.0, 0.0), axis=0)
           + jnp.sum(jnp.where(iota == idx2, 1.0, 0.0), axis=0))
    ps = jnp.sum(p, axis=0)

    @pl.when(i == 0)
    def _init():
        cnt_ref[...] = jnp.zeros_like(cnt_ref)
        psum_ref[...] = jnp.zeros_like(psum_ref)

    cnt_ref[...] += cnt[None, :]
    psum_ref[...] += ps[None, :]

    @pl.when(i == _GRID - 1)
    def _fin():
        aux_ref[...] = (_AUX_COEF * _E) * jnp.sum(
            (cnt_ref[...] / (_T * _TOPK)) * (psum_ref[...] / _T),
            axis=(0, 1), keepdims=True)


def kernel(x, W):
    xt = x.reshape(_T, _DIM)
    wt = W.T
    idx, gates, aux = pl.pallas_call(
        _router_kernel,
        grid=(_GRID,),
        in_specs=[
            pl.BlockSpec((_TBLK, _DIM), lambda i: (i, 0)),
            pl.BlockSpec((_DIM, _E), lambda i: (0, 0)),
        ],
        out_specs=[
            pl.BlockSpec((_TBLK, _TOPK), lambda i: (i, 0)),
            pl.BlockSpec((_TBLK, _TOPK), lambda i: (i, 0)),
            pl.BlockSpec((1, 1), lambda i: (0, 0)),
        ],
        out_shape=[
            jax.ShapeDtypeStruct((_T, _TOPK), jnp.int32),
            jax.ShapeDtypeStruct((_T, _TOPK), jnp.float32),
            jax.ShapeDtypeStruct((1, 1), jnp.float32),
        ],
        scratch_shapes=[
            pltpu.VMEM((1, _E), jnp.float32),
            pltpu.VMEM((1, _E), jnp.float32),
        ],
    )(xt, wt)
    return (idx.reshape(_B, _S, _TOPK),
            gates.reshape(_B, _S, _TOPK),
            aux.reshape(()))


# speedup vs baseline: 1.1504x; 1.1504x over previous
import jax
import jax.numpy as jnp
from jax.experimental import pallas as pl
from jax.experimental.pallas import tpu as pltpu

_B, _S, _DIM = 4, 4096, 2048
_E, _TOPK = 16, 2
_AUX_COEF = 0.01
_T = _B * _S
_TBLK = 1024
_GRID = _T // _TBLK


def _router_kernel(x_ref, w_ref, idx_ref, gate_ref, cnt_ref, psum_ref):
    logits = jax.lax.dot_general(
        x_ref[...], w_ref[...],
        dimension_numbers=(((1,), (0,)), ((), ())),
        preferred_element_type=jnp.float32)
    m = jnp.max(logits, axis=-1, keepdims=True)
    ex = jnp.exp(logits - m)
    z = jnp.sum(ex, axis=-1, keepdims=True)
    p = ex / z

    iota = jax.lax.broadcasted_iota(jnp.int32, p.shape, 1)
    m1 = jnp.max(p, axis=-1, keepdims=True)
    idx1 = jnp.min(jnp.where(p >= m1, iota, _E), axis=-1, keepdims=True)
    pm = jnp.where(iota == idx1, -1.0, p)
    m2 = jnp.max(pm, axis=-1, keepdims=True)
    idx2 = jnp.min(jnp.where(pm >= m2, iota, _E), axis=-1, keepdims=True)

    denom = m1 + m2 + 1e-9
    idx_ref[...] = jnp.concatenate([idx1, idx2], axis=-1)
    gate_ref[...] = jnp.concatenate([m1 / denom, m2 / denom], axis=-1)

    cnt = (jnp.sum(jnp.where(iota == idx1, 1.0, 0.0), axis=0, keepdims=True)
           + jnp.sum(jnp.where(iota == idx2, 1.0, 0.0), axis=0, keepdims=True))
    cnt_ref[...] = cnt[None]
    psum_ref[...] = jnp.sum(p, axis=0, keepdims=True)[None]


def _aux_kernel(cnt_ref, psum_ref, aux_ref):
    cnt = jnp.sum(cnt_ref[...], axis=0, keepdims=True)
    ps = jnp.sum(psum_ref[...], axis=0, keepdims=True)
    aux_ref[...] = (_AUX_COEF * _E) * jnp.sum(
        (cnt / (_T * _TOPK)) * (ps / _T), axis=(0, 1), keepdims=True)


def kernel(x, W):
    xt = x.reshape(_T, _DIM)
    wt = W.T
    idx, gates, cnts, psums = pl.pallas_call(
        _router_kernel,
        grid=(_GRID,),
        in_specs=[
            pl.BlockSpec((_TBLK, _DIM), lambda i: (i, 0)),
            pl.BlockSpec((_DIM, _E), lambda i: (0, 0)),
        ],
        out_specs=[
            pl.BlockSpec((_TBLK, _TOPK), lambda i: (i, 0)),
            pl.BlockSpec((_TBLK, _TOPK), lambda i: (i, 0)),
            pl.BlockSpec((1, 1, _E), lambda i: (i, 0, 0)),
            pl.BlockSpec((1, 1, _E), lambda i: (i, 0, 0)),
        ],
        out_shape=[
            jax.ShapeDtypeStruct((_T, _TOPK), jnp.int32),
            jax.ShapeDtypeStruct((_T, _TOPK), jnp.float32),
            jax.ShapeDtypeStruct((_GRID, 1, _E), jnp.float32),
            jax.ShapeDtypeStruct((_GRID, 1, _E), jnp.float32),
        ],
        compiler_params=pltpu.CompilerParams(
            dimension_semantics=("parallel",),
        ),
    )(xt, wt)
    aux = pl.pallas_call(
        _aux_kernel,
        out_shape=jax.ShapeDtypeStruct((1, 1), jnp.float32),
    )(cnts.reshape(_GRID, _E), psums.reshape(_GRID, _E))
    return (idx.reshape(_B, _S, _TOPK),
            gates.reshape(_B, _S, _TOPK),
            aux.reshape(()))


# speedup vs baseline: 1.2590x; 1.0944x over previous
import jax
import jax.numpy as jnp
from jax.experimental import pallas as pl
from jax.experimental.pallas import tpu as pltpu

_B, _S, _DIM = 4, 4096, 2048
_E, _TOPK = 16, 2
_AUX_COEF = 0.01
_T = _B * _S
_TBLK = 1024
_GRID = _T // _TBLK


def _router_kernel(x_ref, w_ref, idx_ref, gate_ref, cnt_ref, psum_ref):
    logits = jax.lax.dot_general(
        x_ref[...], w_ref[...],
        dimension_numbers=(((1,), (0,)), ((), ())),
        preferred_element_type=jnp.float32)
    lt = logits.T
    m = jnp.max(lt, axis=0, keepdims=True)
    ex = jnp.exp(lt - m)
    z = jnp.sum(ex, axis=0, keepdims=True)
    p = ex / z

    iota = jax.lax.broadcasted_iota(jnp.int32, p.shape, 0)
    m1 = jnp.max(p, axis=0, keepdims=True)
    idx1 = jnp.min(jnp.where(p >= m1, iota, _E), axis=0, keepdims=True)
    hit1 = iota == idx1
    pm = jnp.where(hit1, -1.0, p)
    m2 = jnp.max(pm, axis=0, keepdims=True)
    idx2 = jnp.min(jnp.where(pm >= m2, iota, _E), axis=0, keepdims=True)
    hit2 = iota == idx2

    denom = m1 + m2 + 1e-9
    idx_ref[...] = jnp.concatenate([idx1, idx2], axis=0).T
    gate_ref[...] = jnp.concatenate([m1 / denom, m2 / denom], axis=0).T

    cnt = jnp.sum(jnp.where(hit1 | hit2, 1.0, 0.0), axis=1, keepdims=True)
    cnt_ref[...] = cnt.T[None]
    psum_ref[...] = jnp.sum(p, axis=1, keepdims=True).T[None]


def _aux_kernel(cnt_ref, psum_ref, aux_ref):
    cnt = jnp.sum(cnt_ref[...], axis=0, keepdims=True)
    ps = jnp.sum(psum_ref[...], axis=0, keepdims=True)
    aux_ref[...] = (_AUX_COEF * _E) * jnp.sum(
        (cnt / (_T * _TOPK)) * (ps / _T), axis=(0, 1), keepdims=True)


def kernel(x, W):
    xt = x.reshape(_T, _DIM)
    wt = W.T
    idx, gates, cnts, psums = pl.pallas_call(
        _router_kernel,
        grid=(_GRID,),
        in_specs=[
            pl.BlockSpec((_TBLK, _DIM), lambda i: (i, 0)),
            pl.BlockSpec((_DIM, _E), lambda i: (0, 0)),
        ],
        out_specs=[
            pl.BlockSpec((_TBLK, _TOPK), lambda i: (i, 0)),
            pl.BlockSpec((_TBLK, _TOPK), lambda i: (i, 0)),
            pl.BlockSpec((1, 1, _E), lambda i: (i, 0, 0)),
            pl.BlockSpec((1, 1, _E), lambda i: (i, 0, 0)),
        ],
        out_shape=[
            jax.ShapeDtypeStruct((_T, _TOPK), jnp.int32),
            jax.ShapeDtypeStruct((_T, _TOPK), jnp.float32),
            jax.ShapeDtypeStruct((_GRID, 1, _E), jnp.float32),
            jax.ShapeDtypeStruct((_GRID, 1, _E), jnp.float32),
        ],
        compiler_params=pltpu.CompilerParams(
            dimension_semantics=("parallel",),
        ),
    )(xt, wt)
    aux = pl.pallas_call(
        _aux_kernel,
        out_shape=jax.ShapeDtypeStruct((1, 1), jnp.float32),
    )(cnts.reshape(_GRID, _E), psums.reshape(_GRID, _E))
    return (idx.reshape(_B, _S, _TOPK),
            gates.reshape(_B, _S, _TOPK),
            aux.reshape(()))
